# baseline (speedup 1.0000x reference)
import functools

import jax
import jax.numpy as jnp
from jax import lax
from jax.experimental import pallas as pl
from jax.experimental.pallas import tpu as pltpu
from jax.experimental.pallas import tpu_sc as plsc

_D = 1000
_DA = 896
_DB = _D - _DA
_B = 1024 * 50
_NC = 2
_NS = 16
_NW = _NC * _NS
_BPW = _B // _NW
_CHUNK = 40
_NCHUNK = _BPW // _CHUNK

_mesh = plsc.VectorSubcoreMesh(core_axis_name="c", subcore_axis_name="s")


@functools.partial(
    pl.kernel,
    mesh=_mesh,
    out_type=(
        jax.ShapeDtypeStruct((_B, _D), jnp.float32),
        jax.ShapeDtypeStruct((_B, 128), jnp.float32),
    ),
    scratch_types=[
        pltpu.VMEM((_NCHUNK, _CHUNK), jnp.int32),
        pltpu.VMEM((2, _CHUNK, _DA), jnp.float32),
        pltpu.VMEM((2, _CHUNK, 128), jnp.float32),
        pltpu.SemaphoreType.DMA,
        pltpu.SemaphoreType.DMA,
        pltpu.SemaphoreType.DMA,
        pltpu.SemaphoreType.DMA,
        pltpu.SemaphoreType.DMA,
        pltpu.SemaphoreType.DMA,
        pltpu.SemaphoreType.DMA,
        pltpu.SemaphoreType.DMA,
    ],
)
def _gather_rows(ta_hbm, tb_hbm, idx_hbm, outa_hbm, outb_hbm,
                 idx_v, rows_a, rows_b,
                 ga0, ga1, gb0, gb1, sa0, sa1, sb0, sb1):
    wid = lax.axis_index("s") * _NC + lax.axis_index("c")
    pltpu.sync_copy(idx_hbm.at[wid], idx_v)
    base = wid * _BPW
    gas = (ga0, ga1)
    gbs = (gb0, gb1)
    sas = (sa0, sa1)
    sbs = (sb0, sb1)

    def start_gathers(c, b):
        pltpu.async_copy(ta_hbm.at[idx_v.at[c]], rows_a.at[b], gas[b])
        pltpu.async_copy(tb_hbm.at[idx_v.at[c]], rows_b.at[b], gbs[b])

    def wait_gathers(b):
        pltpu.make_async_copy(ta_hbm.at[pl.ds(0, _CHUNK)], rows_a.at[b],
                              gas[b]).wait()
        pltpu.make_async_copy(tb_hbm.at[pl.ds(0, _CHUNK)], rows_b.at[b],
                              gbs[b]).wait()

    def start_scatters(c, b):
        rows = pl.ds(base + c * _CHUNK, _CHUNK)
        pltpu.async_copy(rows_a.at[b], outa_hbm.at[rows].at[:, pl.ds(0, _DA)],
                         sas[b])
        pltpu.async_copy(rows_b.at[b], outb_hbm.at[rows], sbs[b])

    def wait_scatters(b):
        rows = pl.ds(0, _CHUNK)
        pltpu.make_async_copy(rows_a.at[b],
                              outa_hbm.at[rows].at[:, pl.ds(0, _DA)],
                              sas[b]).wait()
        pltpu.make_async_copy(rows_b.at[b], outb_hbm.at[rows], sbs[b]).wait()

    start_gathers(0, 0)
    start_gathers(1, 1)

    def pair(p, carry):
        c0 = 2 * p
        for b in range(2):
            c = c0 + b
            wait_gathers(b)
            start_scatters(c, b)

            @pl.when(c + 2 < _NCHUNK)
            def _():
                wait_scatters(b)
                start_gathers(c + 2, b)

        return carry

    lax.fori_loop(0, _NCHUNK // 2, pair, 0)
    wait_scatters(0)
    wait_scatters(1)


def kernel(x, table):
    idx = x.reshape(-1).astype(jnp.int32).reshape(_NW, _NCHUNK, _CHUNK)
    table_a = table[:, :_DA]
    table_b = jnp.pad(table[:, _DA:], ((0, 0), (0, 128 - _DB)))
    out_main, out_tail = _gather_rows(table_a, table_b, idx)
    return lax.dynamic_update_slice(out_main, out_tail[:, :_DB], (0, _DA))


# speedup vs baseline: 1.0531x; 1.0531x over previous
import functools

import jax
import jax.numpy as jnp
from jax import lax
from jax.experimental import pallas as pl
from jax.experimental.pallas import tpu as pltpu
from jax.experimental.pallas import tpu_sc as plsc

_D = 1000
_DP = 1024
_B = 1024 * 50
_NC = 2
_NS = 16
_NW = _NC * _NS
_BPW = _B // _NW
_CHUNK = 16
_NBUF = 4
_NCHUNK = _BPW // _CHUNK

_mesh = plsc.VectorSubcoreMesh(core_axis_name="c", subcore_axis_name="s")


@functools.partial(
    pl.kernel,
    mesh=_mesh,
    out_type=jax.ShapeDtypeStruct((_B, _DP), jnp.float32),
    scratch_types=[
        pltpu.VMEM((_NCHUNK, _CHUNK), jnp.int32),
        pltpu.VMEM((_NBUF, _CHUNK, _DP), jnp.float32),
    ] + [pltpu.SemaphoreType.DMA] * (2 * _NBUF),
)
def _gather_rows(t_hbm, idx_hbm, out_hbm, idx_v, rows_v, *sems):
    wid = lax.axis_index("s") * _NC + lax.axis_index("c")
    pltpu.sync_copy(idx_hbm.at[wid], idx_v)
    base = wid * _BPW
    rows = tuple(rows_v.at[b] for b in range(_NBUF))
    gs = sems[:_NBUF]
    ss = sems[_NBUF:]

    def start_gather(c, b):
        pltpu.async_copy(t_hbm.at[idx_v.at[c]], rows[b], gs[b])

    def wait_gather(b):
        pltpu.make_async_copy(t_hbm.at[pl.ds(0, _CHUNK)], rows[b],
                              gs[b]).wait()

    def start_scatter(c, b):
        pltpu.async_copy(rows[b], out_hbm.at[pl.ds(base + c * _CHUNK, _CHUNK)],
                         ss[b])

    def wait_scatter(b):
        pltpu.make_async_copy(rows[b], out_hbm.at[pl.ds(base, _CHUNK)],
                              ss[b]).wait()

    for b in range(_NBUF):
        start_gather(b, b)

    def group(p, carry):
        c0 = _NBUF * p
        for b in range(_NBUF):
            c = c0 + b
            wait_gather(b)
            start_scatter(c, b)

            @pl.when(c + _NBUF < _NCHUNK)
            def _():
                wait_scatter(b)
                start_gather(c + _NBUF, b)

        return carry

    lax.fori_loop(0, _NCHUNK // _NBUF, group, 0)
    for b in range(_NBUF):
        wait_scatter(b)


def kernel(x, table):
    idx = x.reshape(-1).astype(jnp.int32).reshape(_NW, _NCHUNK, _CHUNK)
    table_pad = jnp.pad(table, ((0, 0), (0, _DP - _D)))
    return _gather_rows(table_pad, idx)[:, :_D]


# speedup vs baseline: 1.0561x; 1.0029x over previous
import functools

import jax
import jax.numpy as jnp
from jax import lax
from jax.experimental import pallas as pl
from jax.experimental.pallas import tpu as pltpu
from jax.experimental.pallas import tpu_sc as plsc

_D = 1000
_DP = 1024
_B = 1024 * 50
_NC = 2
_NS = 16
_NW = _NC * _NS
_BPW = _B // _NW
_CHUNK = 32
_NBUF = 2
_NCHUNK = _BPW // _CHUNK

_mesh = plsc.VectorSubcoreMesh(core_axis_name="c", subcore_axis_name="s")


@functools.partial(
    pl.kernel,
    mesh=_mesh,
    out_type=jax.ShapeDtypeStruct((_B, _DP), jnp.float32),
    scratch_types=[
        pltpu.VMEM((_NCHUNK, _CHUNK), jnp.int32),
        pltpu.VMEM((_NBUF, _CHUNK, _DP), jnp.float32),
    ] + [pltpu.SemaphoreType.DMA] * (2 * _NBUF),
)
def _gather_rows(t_hbm, idx_hbm, out_hbm, idx_v, rows_v, *sems):
    wid = lax.axis_index("s") * _NC + lax.axis_index("c")
    pltpu.sync_copy(idx_hbm.at[wid], idx_v)
    base = wid * _BPW
    rows = tuple(rows_v.at[b] for b in range(_NBUF))
    gs = sems[:_NBUF]
    ss = sems[_NBUF:]

    def start_gather(c, b):
        pltpu.async_copy(t_hbm.at[idx_v.at[c]], rows[b], gs[b])

    def wait_gather(b):
        pltpu.make_async_copy(t_hbm.at[pl.ds(0, _CHUNK)], rows[b],
                              gs[b]).wait()

    def start_scatter(c, b):
        pltpu.async_copy(rows[b], out_hbm.at[pl.ds(base + c * _CHUNK, _CHUNK)],
                         ss[b])

    def wait_scatter(b):
        pltpu.make_async_copy(rows[b], out_hbm.at[pl.ds(base, _CHUNK)],
                              ss[b]).wait()

    for b in range(_NBUF):
        start_gather(b, b)

    def group(p, carry):
        c0 = _NBUF * p
        for b in range(_NBUF):
            c = c0 + b
            wait_gather(b)
            start_scatter(c, b)

            @pl.when(c + _NBUF < _NCHUNK)
            def _():
                wait_scatter(b)
                start_gather(c + _NBUF, b)

        return carry

    lax.fori_loop(0, _NCHUNK // _NBUF, group, 0)
    for b in range(_NBUF):
        wait_scatter(b)


def kernel(x, table):
    idx = x.reshape(-1).astype(jnp.int32).reshape(_NW, _NCHUNK, _CHUNK)
    table_pad = jnp.pad(table, ((0, 0), (0, _DP - _D)))
    return _gather_rows(table_pad, idx)[:, :_D]
